# baseline (speedup 1.0000x reference)
import jax
import jax.numpy as jnp
from jax import lax
from jax.experimental import pallas as pl
from jax.experimental.pallas import tpu as pltpu
from jax.experimental.pallas import tpu_sc as plsc

_N = 10000
_E = 320000
_NC = 2
_NS = 16
_NW = _NC * _NS
_RW = 320
_LAST = _N - (_NW - 1) * _RW
_C = 128
_L = 16


def _make_spmm(feat):
    grp = feat // _L
    mesh = plsc.VectorSubcoreMesh(core_axis_name="c", subcore_axis_name="s")

    def body(y_hbm, row_hbm, col_hbm, val_hbm, bnd_hbm, bias_hbm, out_hbm,
             rbuf0, rbuf1, rbuf2, rbuf3, cbuf0, cbuf1, cbuf2, cbuf3,
             vlbuf0, vlbuf1, vlbuf2, vlbuf3,
             gbuf0, gbuf1, accflat, bbuf, bnds, rbbuf, vbuf,
             m0, m1, m2, m3, g0, g1):
        cid = lax.axis_index("c")
        sid = lax.axis_index("s")
        wid = cid * _NS + sid
        row_base = wid * _RW

        pltpu.sync_copy(bnd_hbm, bnds)
        pltpu.sync_copy(bias_hbm, bbuf)
        widv = jnp.full((_L,), wid, jnp.int32)
        b_lo = lax.reduce_max(plsc.load_gather(bnds, [widv]), (0,))
        b_hi = lax.reduce_max(plsc.load_gather(bnds, [widv + 1]), (0,))
        e0 = (b_lo // _C) * _C
        nchunks = (b_hi - e0 + _C - 1) // _C
        niter = (nchunks + 3) // 4

        @pl.loop(0, _RW)
        def _(r):
            for k in range(grp):
                accflat[pl.ds(r * feat + k * _L, _L)] = jnp.zeros(
                    (_L,), jnp.float32)

        rbufs = (rbuf0, rbuf1, rbuf2, rbuf3)
        cbufs = (cbuf0, cbuf1, cbuf2, cbuf3)
        vlbufs = (vlbuf0, vlbuf1, vlbuf2, vlbuf3)
        gbufs = (gbuf0, gbuf1)
        msems = (m0, m1, m2, m3)
        gsems = (g0, g1)
        iota = lax.iota(jnp.int32, _L)
        konst = [iota + k * _L for k in range(grp)]

        def e_of(ch):
            return jnp.minimum(e0 + ch * _C, _E - _C)

        def m_start(ch, m):
            sl = pl.ds(e_of(ch), _C)
            pltpu.async_copy(row_hbm.at[sl], rbufs[m], msems[m])
            pltpu.async_copy(col_hbm.at[sl], cbufs[m], msems[m])
            pltpu.async_copy(val_hbm.at[sl], vlbufs[m], msems[m])

        def m_wait(ch, m):
            sl = pl.ds(e_of(ch), _C)
            pltpu.make_async_copy(row_hbm.at[sl], rbufs[m], msems[m]).wait()
            pltpu.make_async_copy(col_hbm.at[sl], cbufs[m], msems[m]).wait()
            pltpu.make_async_copy(val_hbm.at[sl], vlbufs[m], msems[m]).wait()

        def g_start(m, g):
            pltpu.async_copy(y_hbm.at[cbufs[m]], gbufs[g], gsems[g])

        def g_wait(m, g):
            pltpu.make_async_copy(y_hbm.at[cbufs[m]], gbufs[g],
                                  gsems[g]).wait()

        def compute(ch, m, g):
            gb = gbufs[g]
            e_i = e_of(ch)
            live = ch < nchunks

            @pl.loop(0, _C // _L)
            def _(j0):
                row16 = rbufs[m][pl.ds(j0 * _L, _L)]
                val16 = vlbufs[m][pl.ds(j0 * _L, _L)]
                g16 = (e_i + j0 * _L) + iota
                msk = (g16 >= b_lo) & (g16 < b_hi) & live
                v16 = jnp.where(msk, val16, jnp.float32(0.0))
                rl16 = jnp.clip(row16 - row_base, 0, _RW - 1)
                vbuf[pl.ds(j0 * _L, _L)] = v16
                rbbuf[pl.ds(j0 * _L, _L)] = rl16 * feat

            @plsc.parallel_loop(0, _C, unroll=16)
            def _(j):
                jsplat = jnp.zeros((_L,), jnp.int32) + j
                rsp = plsc.load_gather(rbbuf, [jsplat])
                vsp = plsc.load_gather(vbuf, [jsplat])
                for k in range(grp):
                    s = gb[j, pl.ds(k * _L, _L)]
                    plsc.addupdate_scatter(
                        accflat, [rsp + konst[k]], s * vsp)

        @pl.when(nchunks > 0)
        def _():
            for r in range(4):
                m_start(r, r)
            m_wait(0, 0)
            g_start(0, 0)
            m_wait(1, 1)
            g_start(1, 1)

            @pl.loop(0, niter)
            def _(i4):
                base = 4 * i4
                for r in range(4):
                    ch = base + r
                    g_wait(r, r % 2)
                    compute(ch, r, r % 2)
                    m_start(ch + 4, r)
                    m_wait(ch + 2, (r + 2) % 4)
                    g_start((r + 2) % 4, r % 2)

            g_wait(0, 0)
            g_wait(1, 1)
            m_wait(0, 2)
            m_wait(0, 3)

        @pl.loop(0, _RW)
        def _(r):
            for k in range(grp):
                sl = pl.ds(r * feat + k * _L, _L)
                accflat[sl] = accflat[sl] + bbuf[pl.ds(k * _L, _L)]

        @pl.when(wid < _NW - 1)
        def _():
            pltpu.sync_copy(
                accflat, out_hbm.at[pl.ds(row_base * feat, _RW * feat)])

        @pl.when(wid == _NW - 1)
        def _():
            pltpu.sync_copy(
                accflat.at[pl.ds(0, _LAST * feat)],
                out_hbm.at[pl.ds(row_base * feat, _LAST * feat)])

    return pl.kernel(
        body,
        out_type=jax.ShapeDtypeStruct((_N * feat,), jnp.float32),
        mesh=mesh,
        scratch_types=[
            pltpu.VMEM((_C,), jnp.int32),
            pltpu.VMEM((_C,), jnp.int32),
            pltpu.VMEM((_C,), jnp.int32),
            pltpu.VMEM((_C,), jnp.int32),
            pltpu.VMEM((_C,), jnp.int32),
            pltpu.VMEM((_C,), jnp.int32),
            pltpu.VMEM((_C,), jnp.int32),
            pltpu.VMEM((_C,), jnp.int32),
            pltpu.VMEM((_C,), jnp.float32),
            pltpu.VMEM((_C,), jnp.float32),
            pltpu.VMEM((_C,), jnp.float32),
            pltpu.VMEM((_C,), jnp.float32),
            pltpu.VMEM((_C, feat), jnp.float32),
            pltpu.VMEM((_C, feat), jnp.float32),
            pltpu.VMEM((_RW * feat,), jnp.float32),
            pltpu.VMEM((feat,), jnp.float32),
            pltpu.VMEM((_NW + 16,), jnp.int32),
            pltpu.VMEM((_C,), jnp.int32),
            pltpu.VMEM((_C,), jnp.float32),
            pltpu.SemaphoreType.DMA,
            pltpu.SemaphoreType.DMA,
            pltpu.SemaphoreType.DMA,
            pltpu.SemaphoreType.DMA,
            pltpu.SemaphoreType.DMA,
            pltpu.SemaphoreType.DMA,
        ],
        compiler_params=pltpu.CompilerParams(
            use_tc_tiling_on_sc=False, needs_layout_passes=False),
    )


_spmm128 = _make_spmm(128)
_spmm64 = _make_spmm(64)

_BM = 2000


def _mm1(x, w1):
    def body(x_ref, w_ref, o_ref):
        o_ref[...] = lax.dot_general(
            x_ref[...], w_ref[...], (((1,), (1,)), ((), ())),
            preferred_element_type=jnp.float32)

    return pl.pallas_call(
        body,
        grid=(_N // _BM,),
        in_specs=[pl.BlockSpec((_BM, 128), lambda i: (i, 0)),
                  pl.BlockSpec((128, 128), lambda i: (0, 0))],
        out_specs=pl.BlockSpec((_BM, 128), lambda i: (i, 0)),
        out_shape=jax.ShapeDtypeStruct((_N, 128), jnp.float32),
    )(x, w1)


def _mm2(z, w2):
    def body(z_ref, w_ref, o_ref):
        o_ref[...] = lax.dot_general(
            z_ref[...], w_ref[...], (((1,), (1,)), ((), ())),
            preferred_element_type=jnp.float32)

    return pl.pallas_call(
        body,
        grid=(_N // _BM,),
        in_specs=[pl.BlockSpec((_BM, 128), lambda i: (i, 0)),
                  pl.BlockSpec((64, 128), lambda i: (0, 0))],
        out_specs=pl.BlockSpec((_BM, 64), lambda i: (i, 0)),
        out_shape=jax.ShapeDtypeStruct((_N, 64), jnp.float32),
    )(z, w2)


def kernel(x, adj_row, adj_col, adj_val, W1, b1, W2, b2):
    bnd = jnp.searchsorted(
        adj_row, jnp.arange(_NW + 1, dtype=jnp.int32) * _RW).astype(jnp.int32)
    bnd = jnp.concatenate([bnd, jnp.full((15,), _E, jnp.int32)])
    y = _mm1(x, W1)
    z = _spmm128(y, adj_row, adj_col, adj_val, bnd, b1).reshape(_N, 128)
    u = _mm2(z, W2)
    return _spmm64(u, adj_row, adj_col, adj_val, bnd, b2).reshape(_N, 64)


# speedup vs baseline: 1.2765x; 1.2765x over previous
import jax
import jax.numpy as jnp
from jax import lax
from jax.experimental import pallas as pl
from jax.experimental.pallas import tpu as pltpu
from jax.experimental.pallas import tpu_sc as plsc

_N = 10000
_E = 320000
_NC = 2
_NS = 16
_NW = _NC * _NS
_RW = 320
_LAST = _N - (_NW - 1) * _RW
_C = 128
_L = 16


def _make_spmm(feat):
    grp = feat // _L
    mesh = plsc.VectorSubcoreMesh(core_axis_name="c", subcore_axis_name="s")

    def body(y_hbm, row_hbm, col_hbm, val_hbm, bnd_hbm, bias_hbm, out_hbm,
             rbuf0, rbuf1, rbuf2, rbuf3, cbuf0, cbuf1, cbuf2, cbuf3,
             vlbuf0, vlbuf1, vlbuf2, vlbuf3,
             gbuf0, gbuf1, accflat, bbuf, bnds, rbbuf, vbuf,
             m0, m1, m2, m3, g0, g1):
        cid = lax.axis_index("c")
        sid = lax.axis_index("s")
        wid = cid * _NS + sid
        row_base = wid * _RW

        pltpu.sync_copy(bnd_hbm, bnds)
        pltpu.sync_copy(bias_hbm, bbuf)
        widv = jnp.full((_L,), wid, jnp.int32)
        b_lo = lax.reduce_max(plsc.load_gather(bnds, [widv]), (0,))
        b_hi = lax.reduce_max(plsc.load_gather(bnds, [widv + 1]), (0,))
        e0 = (b_lo // _C) * _C
        nchunks = (b_hi - e0 + _C - 1) // _C
        niter = (nchunks + 3) // 4

        @pl.loop(0, _RW)
        def _(r):
            for k in range(grp):
                accflat[pl.ds(r * feat + k * _L, _L)] = jnp.zeros(
                    (_L,), jnp.float32)

        rbufs = (rbuf0, rbuf1, rbuf2, rbuf3)
        cbufs = (cbuf0, cbuf1, cbuf2, cbuf3)
        vlbufs = (vlbuf0, vlbuf1, vlbuf2, vlbuf3)
        gbufs = (gbuf0, gbuf1)
        msems = (m0, m1, m2, m3)
        gsems = (g0, g1)
        iota = lax.iota(jnp.int32, _L)
        konst = [iota + k * _L for k in range(grp)]

        def e_of(ch):
            return jnp.minimum(e0 + ch * _C, _E - _C)

        def m_start(ch, m):
            sl = pl.ds(e_of(ch), _C)
            pltpu.async_copy(row_hbm.at[sl], rbufs[m], msems[m])
            pltpu.async_copy(col_hbm.at[sl], cbufs[m], msems[m])
            pltpu.async_copy(val_hbm.at[sl], vlbufs[m], msems[m])

        def m_wait(ch, m):
            sl = pl.ds(e_of(ch), _C)
            pltpu.make_async_copy(row_hbm.at[sl], rbufs[m], msems[m]).wait()
            pltpu.make_async_copy(col_hbm.at[sl], cbufs[m], msems[m]).wait()
            pltpu.make_async_copy(val_hbm.at[sl], vlbufs[m], msems[m]).wait()

        def g_start(m, g):
            pltpu.async_copy(y_hbm.at[cbufs[m]], gbufs[g], gsems[g])

        def g_wait(m, g):
            pltpu.make_async_copy(y_hbm.at[cbufs[m]], gbufs[g],
                                  gsems[g]).wait()

        def compute(ch, m, g):
            gb = gbufs[g]
            e_i = e_of(ch)
            live = ch < nchunks

            @pl.loop(0, _C // _L)
            def _(j0):
                row16 = rbufs[m][pl.ds(j0 * _L, _L)]
                val16 = vlbufs[m][pl.ds(j0 * _L, _L)]
                g16 = (e_i + j0 * _L) + iota
                msk = (g16 >= b_lo) & (g16 < b_hi) & live
                v16 = jnp.where(msk, val16, jnp.float32(0.0))
                rl16 = jnp.clip(row16 - row_base, 0, _RW - 1)
                vbuf[pl.ds(j0 * _L, _L)] = v16
                rbbuf[pl.ds(j0 * _L, _L)] = rl16 * feat

            @plsc.parallel_loop(0, _C, unroll=8)
            def _(j):
                jsplat = jnp.zeros((_L,), jnp.int32) + j
                rsp = plsc.load_gather(rbbuf, [jsplat])
                vsp = plsc.load_gather(vbuf, [jsplat])
                for k in range(grp):
                    s = gb[j, pl.ds(k * _L, _L)]
                    plsc.addupdate_scatter(
                        accflat, [rsp + konst[k]], s * vsp)

        @pl.when(nchunks > 0)
        def _():
            for r in range(4):
                m_start(r, r)
            m_wait(0, 0)
            g_start(0, 0)
            m_wait(1, 1)
            g_start(1, 1)

            @pl.loop(0, niter)
            def _(i4):
                base = 4 * i4
                for r in range(4):
                    ch = base + r
                    g_wait(r, r % 2)
                    compute(ch, r, r % 2)
                    m_start(ch + 4, r)
                    m_wait(ch + 2, (r + 2) % 4)
                    g_start((r + 2) % 4, r % 2)

            g_wait(0, 0)
            g_wait(1, 1)
            m_wait(0, 2)
            m_wait(0, 3)

        @pl.loop(0, _RW)
        def _(r):
            for k in range(grp):
                sl = pl.ds(r * feat + k * _L, _L)
                accflat[sl] = accflat[sl] + bbuf[pl.ds(k * _L, _L)]

        @pl.when(wid < _NW - 1)
        def _():
            pltpu.sync_copy(
                accflat, out_hbm.at[pl.ds(row_base * feat, _RW * feat)])

        @pl.when(wid == _NW - 1)
        def _():
            pltpu.sync_copy(
                accflat.at[pl.ds(0, _LAST * feat)],
                out_hbm.at[pl.ds(row_base * feat, _LAST * feat)])

    return pl.kernel(
        body,
        out_type=jax.ShapeDtypeStruct((_N * feat,), jnp.float32),
        mesh=mesh,
        scratch_types=[
            pltpu.VMEM((_C,), jnp.int32),
            pltpu.VMEM((_C,), jnp.int32),
            pltpu.VMEM((_C,), jnp.int32),
            pltpu.VMEM((_C,), jnp.int32),
            pltpu.VMEM((_C,), jnp.int32),
            pltpu.VMEM((_C,), jnp.int32),
            pltpu.VMEM((_C,), jnp.int32),
            pltpu.VMEM((_C,), jnp.int32),
            pltpu.VMEM((_C,), jnp.float32),
            pltpu.VMEM((_C,), jnp.float32),
            pltpu.VMEM((_C,), jnp.float32),
            pltpu.VMEM((_C,), jnp.float32),
            pltpu.VMEM((_C, feat), jnp.float32),
            pltpu.VMEM((_C, feat), jnp.float32),
            pltpu.VMEM((_RW * feat,), jnp.float32),
            pltpu.VMEM((feat,), jnp.float32),
            pltpu.VMEM((_NW + 16,), jnp.int32),
            pltpu.VMEM((_C,), jnp.int32),
            pltpu.VMEM((_C,), jnp.float32),
            pltpu.SemaphoreType.DMA,
            pltpu.SemaphoreType.DMA,
            pltpu.SemaphoreType.DMA,
            pltpu.SemaphoreType.DMA,
            pltpu.SemaphoreType.DMA,
            pltpu.SemaphoreType.DMA,
        ],
        compiler_params=pltpu.CompilerParams(
            use_tc_tiling_on_sc=False, needs_layout_passes=False),
    )


_spmm128 = _make_spmm(128)
_spmm64 = _make_spmm(64)

_BM = 2000


def _mm1(x, w1):
    def body(x_ref, w_ref, o_ref):
        o_ref[...] = lax.dot_general(
            x_ref[...], w_ref[...], (((1,), (1,)), ((), ())),
            preferred_element_type=jnp.float32)

    return pl.pallas_call(
        body,
        grid=(_N // _BM,),
        in_specs=[pl.BlockSpec((_BM, 128), lambda i: (i, 0)),
                  pl.BlockSpec((128, 128), lambda i: (0, 0))],
        out_specs=pl.BlockSpec((_BM, 128), lambda i: (i, 0)),
        out_shape=jax.ShapeDtypeStruct((_N, 128), jnp.float32),
    )(x, w1)


def _mm2(z, w2):
    def body(z_ref, w_ref, o_ref):
        o_ref[...] = lax.dot_general(
            z_ref[...], w_ref[...], (((1,), (1,)), ((), ())),
            preferred_element_type=jnp.float32)

    return pl.pallas_call(
        body,
        grid=(_N // _BM,),
        in_specs=[pl.BlockSpec((_BM, 128), lambda i: (i, 0)),
                  pl.BlockSpec((64, 128), lambda i: (0, 0))],
        out_specs=pl.BlockSpec((_BM, 64), lambda i: (i, 0)),
        out_shape=jax.ShapeDtypeStruct((_N, 64), jnp.float32),
    )(z, w2)


def kernel(x, adj_row, adj_col, adj_val, W1, b1, W2, b2):
    bnd = jnp.searchsorted(
        adj_row, jnp.arange(_NW + 1, dtype=jnp.int32) * _RW).astype(jnp.int32)
    bnd = jnp.concatenate([bnd, jnp.full((15,), _E, jnp.int32)])
    y = _mm1(x, W1)
    z = _spmm128(y, adj_row, adj_col, adj_val, bnd, b1).reshape(_N, 128)
    u = _mm2(z, W2)
    return _spmm64(u, adj_row, adj_col, adj_val, bnd, b2).reshape(_N, 64)
